# Initial kernel scaffold; baseline (speedup 1.0000x reference)
import jax
import jax.numpy as jnp
from jax.experimental import pallas as pl


def kernel(edge_index, edge_vals, uEmbeds, iEmbeds, keepRate):
    raise NotImplementedError("write your pallas kernel here")


# speedup vs baseline: 2.2159x; 2.2159x over previous
import functools

import jax
import jax.numpy as jnp
from jax import lax
from jax.experimental import pallas as pl
from jax.experimental.pallas import tpu as pltpu
from jax.experimental.pallas import tpu_sc as plsc

N = 10000
USER = 5000
D = 128
E = 320000
NC = 2
NS = 16
NW = NC * NS
C = 128
CHUNKS = 80
EPT = C * CHUNKS
EP = EPT * NW
RPT = 624
TAIL = N - RPT * NS
TAIL_OFF = RPT * NS


def _spmm_body(rows_hbm, cols_hbm, vals_hbm, x_hbm, y0_hbm, y1_hbm,
               colbuf, rowbuf, valbuf, gbuf, y_sh, sem):
    cid = lax.axis_index("c")
    sid = lax.axis_index("s")
    wid = cid * NS + sid

    def zrow(i, carry):
        for j in range(D // 16):
            gbuf[i, pl.ds(j * 16, 16)] = jnp.zeros((16,), jnp.float32)
        return carry
    lax.fori_loop(0, C, zrow, 0)
    base_r = sid * RPT
    for k in range(RPT // C):
        pltpu.sync_copy(gbuf, y_sh.at[pl.ds(base_r + k * C, C)])
    rem = RPT % C
    pltpu.sync_copy(gbuf.at[pl.ds(0, rem)],
                    y_sh.at[pl.ds(base_r + RPT - rem, rem)])

    @pl.when(sid == 0)
    def _():
        pltpu.sync_copy(gbuf.at[pl.ds(0, TAIL)], y_sh.at[pl.ds(TAIL_OFF, TAIL)])

    plsc.subcore_barrier()

    base = wid * EPT

    def chunk(k, carry):
        off = base + k * C
        pltpu.sync_copy(cols_hbm.at[pl.ds(off, C)], colbuf)
        pltpu.sync_copy(rows_hbm.at[pl.ds(off, C)], rowbuf)
        pltpu.sync_copy(vals_hbm.at[pl.ds(off, C)], valbuf)
        pltpu.async_copy(x_hbm.at[colbuf], gbuf, sem).wait()

        def scale(g, c2):
            vv = valbuf[pl.ds(g * 16, 16)]
            for i in range(16):
                v = vv[i]
                e = g * 16 + i
                for j in range(D // 16):
                    sl = pl.ds(j * 16, 16)
                    gbuf[e, sl] = gbuf[e, sl] * v
            return c2
        lax.fori_loop(0, C // 16, scale, 0)

        pltpu.sync_copy(gbuf, y_sh.at[rowbuf], add=True)
        return carry
    lax.fori_loop(0, CHUNKS, chunk, 0)
    plsc.subcore_barrier()

    sl = pl.ds(sid * RPT, RPT)
    tl = pl.ds(TAIL_OFF, TAIL)

    @pl.when(cid == 0)
    def _():
        pltpu.sync_copy(y_sh.at[sl], y0_hbm.at[sl])

        @pl.when(sid == 0)
        def _():
            pltpu.sync_copy(y_sh.at[tl], y0_hbm.at[tl])

    @pl.when(cid == 1)
    def _():
        pltpu.sync_copy(y_sh.at[sl], y1_hbm.at[sl])

        @pl.when(sid == 0)
        def _():
            pltpu.sync_copy(y_sh.at[tl], y1_hbm.at[tl])


_spmm = pl.kernel(
    _spmm_body,
    out_type=(jax.ShapeDtypeStruct((N, D), jnp.float32),) * 2,
    mesh=plsc.VectorSubcoreMesh(core_axis_name="c", subcore_axis_name="s",
                                num_cores=NC, num_subcores=NS),
    scratch_types=[
        pltpu.VMEM((C,), jnp.int32),
        pltpu.VMEM((C,), jnp.int32),
        pltpu.VMEM((C,), jnp.float32),
        pltpu.VMEM((C, D), jnp.float32),
        pltpu.VMEM_SHARED((N, D), jnp.float32),
        pltpu.SemaphoreType.DMA,
    ],
)


def _combine_body(a_ref, b_ref, t_ref, x_out, t_out):
    s = a_ref[...] + b_ref[...]
    x_out[...] = s
    t_out[...] = t_ref[...] + s


_combine = pl.pallas_call(
    _combine_body,
    grid=(10,),
    in_specs=[pl.BlockSpec((N // 10, D), lambda i: (i, 0))] * 3,
    out_specs=[pl.BlockSpec((N // 10, D), lambda i: (i, 0))] * 2,
    out_shape=(jax.ShapeDtypeStruct((N, D), jnp.float32),) * 2,
)


def kernel(edge_index, edge_vals, uEmbeds, iEmbeds, keepRate):
    rows = edge_index[0]
    cols = edge_index[1]
    pad = EP - E
    rows_p = jnp.concatenate([rows, jnp.zeros((pad,), jnp.int32)])
    cols_p = jnp.concatenate([cols, jnp.zeros((pad,), jnp.int32)])
    vals_p = jnp.concatenate([edge_vals, jnp.zeros((pad,), jnp.float32)])
    x0 = jnp.concatenate([uEmbeds, iEmbeds], axis=0)

    y0a, y0b = _spmm(rows_p, cols_p, vals_p, x0)
    x1, t1 = _combine(y0a, y0b, x0)
    y1a, y1b = _spmm(rows_p, cols_p, vals_p, x1)
    x2, t2 = _combine(y1a, y1b, t1)
    y2a, y2b = _spmm(rows_p, cols_p, vals_p, x2)
    _x3, t3 = _combine(y2a, y2b, t2)
    return t3[:USER], t3[USER:]
